# baseline (speedup 1.0000x reference)
import functools

import jax
import jax.numpy as jnp
from jax import lax
from jax.experimental import pallas as pl
from jax.experimental.pallas import tpu as pltpu
from jax.experimental.pallas import tpu_sc as plsc

N = 10000
D_IN = 128
D_H = 256
H = 64
NP = 10240
NPA = 10112
CHUNK = 256
E_PAD = 327680
ROWS = E_PAD // 128
CPT_AGG = ROWS // 16
CPT_DEG = ROWS // 32
STRIPE = NP // 16

_MESH = plsc.VectorSubcoreMesh(core_axis_name="c", subcore_axis_name="s",
                               num_cores=2)
_SC_PARAMS = pltpu.CompilerParams(use_tc_tiling_on_sc=False)
_HIGH = jax.lax.Precision.DEFAULT


@functools.partial(
    pl.kernel,
    out_type=jax.ShapeDtypeStruct((2, NP, 16), jnp.float32),
    mesh=_MESH,
    compiler_params=_SC_PARAMS,
    scratch_types=[
        pltpu.VMEM((CPT_DEG, 128), jnp.int32),
        pltpu.VMEM((128, 16), jnp.float32),
        pltpu.VMEM_SHARED((NP, 16), jnp.float32),
    ],
)
def _deg_sc(dst2d, zeros16, ones16, out, idx_v, stage_v, acc):
    c = lax.axis_index("c")
    s = lax.axis_index("s")
    w = c * 16 + s
    pltpu.sync_copy(zeros16, stage_v)
    for k in range(STRIPE // 128):
        pltpu.sync_copy(stage_v, acc.at[pl.ds(s * STRIPE + k * 128, 128)])
    pltpu.sync_copy(dst2d.at[pl.ds(w * CPT_DEG, CPT_DEG)], idx_v)
    plsc.subcore_barrier()
    pltpu.sync_copy(ones16, stage_v)

    def body(j, carry):
        pltpu.sync_copy(stage_v, acc.at[idx_v.at[j]], add=True)
        return carry

    lax.fori_loop(0, CPT_DEG, body, 0)
    plsc.subcore_barrier()
    pltpu.sync_copy(acc.at[pl.ds(s * STRIPE, STRIPE)],
                    out.at[c].at[pl.ds(s * STRIPE, STRIPE)])


_NBUF = 4
_DEPTH = 2
_CH = 64
_NCHP = 80
_PIECES = 4
_ASTRIPE = NPA // 16


@functools.partial(
    pl.kernel,
    out_type=jax.ShapeDtypeStruct((2, NPA, 128), jnp.float32),
    mesh=_MESH,
    compiler_params=_SC_PARAMS,
    scratch_types=[
        pltpu.VMEM((_NCHP, _CH), jnp.int32),
        pltpu.VMEM((_NCHP, _CH), jnp.int32),
        [pltpu.VMEM((_CH, 128), jnp.float32)] * _NBUF,
        [pltpu.SemaphoreType.DMA] * _NBUF,
        [pltpu.SemaphoreType.DMA] * _NBUF,
        pltpu.VMEM_SHARED((NPA, 128), jnp.float32),
    ],
)
def _agg_sc(y_st, src64, dst64, zeros_c, out, src_v, dst_v, sts, sgs,
            sss, acc):
    c = lax.axis_index("c")
    s = lax.axis_index("s")
    pltpu.sync_copy(zeros_c, sts[0])
    for k in range(9):
        pltpu.sync_copy(sts[0], acc.at[pl.ds(s * _ASTRIPE + k * _CH, _CH)])
    pltpu.sync_copy(sts[0].at[pl.ds(0, 56)],
                    acc.at[pl.ds(s * _ASTRIPE + 576, 56)])
    plsc.subcore_barrier()
    ytab = y_st.at[c]

    def wait1(buf, sem):
        pltpu.make_async_copy(zeros_c, buf, sem).wait()

    for pc in range(_PIECES):
        base = s * (_PIECES * _NCHP) + pc * _NCHP
        pltpu.sync_copy(src64.at[pl.ds(base, _NCHP)], src_v)
        pltpu.sync_copy(dst64.at[pl.ds(base, _NCHP)], dst_v)
        for b in range(_DEPTH):
            pltpu.async_copy(ytab.at[src_v.at[b]], sts[b], sgs[b])

        def body(i, carry):
            for t in range(_NBUF):
                j = i * _NBUF + t
                wait1(sts[t], sgs[t])
                pltpu.async_copy(sts[t], acc.at[dst_v.at[j]],
                                 sss[t], add=True)
                b2 = (t + _DEPTH) % _NBUF

                @pl.when(j >= _NBUF - _DEPTH)
                def _drain():
                    wait1(sts[b2], sss[b2])

                @pl.when(j + _DEPTH < _NCHP)
                def _refill():
                    pltpu.async_copy(ytab.at[src_v.at[j + _DEPTH]],
                                     sts[b2], sgs[b2])

            return carry

        _NFULL = _NCHP // _NBUF
        lax.fori_loop(0, _NFULL, body, 0)
        for j in range(_NFULL * _NBUF, _NCHP):
            t = j % _NBUF
            wait1(sts[t], sgs[t])
            pltpu.async_copy(sts[t], acc.at[dst_v.at[j]], sss[t], add=True)
            wait1(sts[(t + _DEPTH) % _NBUF], sss[(t + _DEPTH) % _NBUF])
        for k in range(_NCHP - (_NBUF - _DEPTH), _NCHP):
            wait1(sts[k % _NBUF], sss[k % _NBUF])
    plsc.subcore_barrier()
    pltpu.sync_copy(acc.at[pl.ds(s * _ASTRIPE, _ASTRIPE)],
                    out.at[c].at[pl.ds(s * _ASTRIPE, _ASTRIPE)])


_BLK = 400
_GRID = N // _BLK


def _dinv_of(d0, d1):
    deg = d0[:, :1] + d1[:, :1] + 1.0
    return lax.rsqrt(deg)


def _tc1_body(x_ref, w_ref, d0_ref, d1_ref, y_ref):
    dinv = _dinv_of(d0_ref[...], d1_ref[...])
    y = jnp.dot(x_ref[...], w_ref[...], precision=_HIGH,
                preferred_element_type=jnp.float32) * dinv
    y_ref[0] = y[:, :128]
    y_ref[1] = y[:, 128:]


def _ln_lrelu(pre, g, be):
    m = jnp.mean(pre, axis=1, keepdims=True)
    cz = pre - m
    var = jnp.mean(cz * cz, axis=1, keepdims=True)
    ln = cz * lax.rsqrt(var + 1e-5) * g + be
    return jnp.where(ln >= 0, ln, 0.2 * ln)


def _tc2_body(a_ref, y_ref, d0_ref, d1_ref, b_ref, g_ref, be_ref, w2_ref,
              out_ref):
    dinv = _dinv_of(d0_ref[...], d1_ref[...])
    agg = jnp.concatenate([a_ref[0], a_ref[1]], axis=1)
    y = jnp.concatenate([y_ref[0], y_ref[1]], axis=1)
    pre = (agg + y) * dinv + b_ref[...]
    h = _ln_lrelu(pre, g_ref[...], be_ref[...])
    xw = jnp.dot(h, w2_ref[...], precision=_HIGH,
                 preferred_element_type=jnp.float32) * dinv
    out_ref[0] = xw[:, :128]
    out_ref[1] = xw[:, 128:]


def _tc3_body(a_ref, y_ref, d0_ref, d1_ref, b_ref, g_ref, be_ref,
              wa1_ref, ba1_ref, wa2_ref, ba2_ref, h_ref, s_ref):
    dinv = _dinv_of(d0_ref[...], d1_ref[...])
    agg = jnp.concatenate([a_ref[0], a_ref[1]], axis=1)
    y = jnp.concatenate([y_ref[0], y_ref[1]], axis=1)
    pre = (agg + y) * dinv + b_ref[...]
    h = _ln_lrelu(pre, g_ref[...], be_ref[...])
    h_ref[...] = h
    t = jnp.tanh(jnp.dot(h, wa1_ref[...], precision=_HIGH,
                         preferred_element_type=jnp.float32) + ba1_ref[...])
    s_ref[...] = jnp.dot(t, wa2_ref[...], precision=_HIGH,
                         preferred_element_type=jnp.float32) + ba2_ref[...]


def _tc4_body(s_ref, h_ref, att_ref, gf_ref):
    sc = s_ref[...]
    m = jnp.max(sc)
    e = jnp.exp(sc - m)
    z = jnp.sum(e)
    att = e / z
    att_ref[...] = att
    gf_ref[...] = lax.dot_general(att, h_ref[...], (((0,), (0,)), ((), ())),
                                  precision=_HIGH,
                                  preferred_element_type=jnp.float32)


def _row_spec(width):
    return pl.BlockSpec((_BLK, width), lambda i: (i, 0))


def _full_spec(shape):
    nd = len(shape)
    return pl.BlockSpec(shape, lambda i: (0,) * nd)


def _split_spec():
    return pl.BlockSpec((2, _BLK, 128), lambda i: (0, i, 0))


def kernel(x, edge_index, W1, b1, g1, be1, W2, b2, g2, be2, Wa1, ba1, Wa2, ba2):
    e = edge_index.shape[1]
    pad = E_PAD - e
    src = edge_index[0].astype(jnp.int32)
    dst = edge_index[1].astype(jnp.int32)
    src_p = jnp.concatenate([src, jnp.zeros((pad,), jnp.int32)])
    dst_p = jnp.concatenate([dst, jnp.full((pad,), N, jnp.int32)])
    src64 = src_p.reshape(E_PAD // 64, 64)
    dst64 = dst_p.reshape(E_PAD // 64, 64)
    dst2d = dst_p.reshape(ROWS, 128)
    zeros16 = jnp.zeros((128, 16), jnp.float32)
    ones16 = jnp.ones((128, 16), jnp.float32)
    zeros_c = jnp.zeros((64, 128), jnp.float32)
    b1r, g1r, be1r = b1.reshape(1, -1), g1.reshape(1, -1), be1.reshape(1, -1)
    b2r, g2r, be2r = b2.reshape(1, -1), g2.reshape(1, -1), be2.reshape(1, -1)
    ba1r, ba2r = ba1.reshape(1, -1), ba2.reshape(1, -1)

    deg2 = _deg_sc(dst2d, zeros16, ones16)
    deg0 = deg2[0, :N]
    deg1 = deg2[1, :N]

    y1 = pl.pallas_call(
        _tc1_body,
        grid=(_GRID,),
        in_specs=[_row_spec(D_IN), _full_spec((D_IN, D_H)),
                  _row_spec(16), _row_spec(16)],
        out_specs=_split_spec(),
        out_shape=jax.ShapeDtypeStruct((2, N, 128), jnp.float32),
    )(x, W1, deg0, deg1)

    agg1 = _agg_sc(y1, src64, dst64, zeros_c)

    y2 = pl.pallas_call(
        _tc2_body,
        grid=(_GRID,),
        in_specs=[_split_spec(), _split_spec(), _row_spec(16), _row_spec(16),
                  _full_spec((1, D_H)), _full_spec((1, D_H)),
                  _full_spec((1, D_H)), _full_spec((D_H, D_H))],
        out_specs=_split_spec(),
        out_shape=jax.ShapeDtypeStruct((2, N, 128), jnp.float32),
    )(agg1, y1, deg0, deg1, b1r, g1r, be1r, W2)

    agg2 = _agg_sc(y2, src64, dst64, zeros_c)

    h2, s = pl.pallas_call(
        _tc3_body,
        grid=(_GRID,),
        in_specs=[_split_spec(), _split_spec(), _row_spec(16), _row_spec(16),
                  _full_spec((1, D_H)), _full_spec((1, D_H)),
                  _full_spec((1, D_H)), _full_spec((D_H, D_H)),
                  _full_spec((1, D_H)), _full_spec((D_H, 1)),
                  _full_spec((1, 1))],
        out_specs=[_row_spec(D_H), _row_spec(1)],
        out_shape=[jax.ShapeDtypeStruct((N, D_H), jnp.float32),
                   jax.ShapeDtypeStruct((N, 1), jnp.float32)],
    )(agg2, y2, deg0, deg1, b2r, g2r, be2r, Wa1, ba1r, Wa2, ba2r)

    att, gf = pl.pallas_call(
        _tc4_body,
        grid=(1,),
        in_specs=[_full_spec((N, 1)), _full_spec((N, D_H))],
        out_specs=[_full_spec((N, 1)), _full_spec((1, D_H))],
        out_shape=[jax.ShapeDtypeStruct((N, 1), jnp.float32),
                   jax.ShapeDtypeStruct((1, D_H), jnp.float32)],
    )(s, h2)

    return gf, att.reshape(N)


# speedup vs baseline: 1.0429x; 1.0429x over previous
import functools

import jax
import jax.numpy as jnp
from jax import lax
from jax.experimental import pallas as pl
from jax.experimental.pallas import tpu as pltpu
from jax.experimental.pallas import tpu_sc as plsc

N = 10000
D_IN = 128
D_H = 256
NP = 10240
NPA = 10112
E_PAD = 327680
ROWS = E_PAD // 128
CPT_DEG = ROWS // 32
STRIPE = NP // 16

_MESH = plsc.VectorSubcoreMesh(core_axis_name="c", subcore_axis_name="s",
                               num_cores=2)
_SC_PARAMS = pltpu.CompilerParams(use_tc_tiling_on_sc=False)
_HIGH = jax.lax.Precision.DEFAULT


@functools.partial(
    pl.kernel,
    out_type=jax.ShapeDtypeStruct((2, NP, 16), jnp.float32),
    mesh=_MESH,
    compiler_params=_SC_PARAMS,
    scratch_types=[
        pltpu.VMEM((CPT_DEG, 128), jnp.int32),
        pltpu.VMEM((128, 16), jnp.float32),
        pltpu.VMEM_SHARED((NP, 16), jnp.float32),
    ],
)
def _deg_sc(dst2d, zeros16, ones16, out, idx_v, stage_v, acc):
    c = lax.axis_index("c")
    s = lax.axis_index("s")
    w = c * 16 + s
    pltpu.sync_copy(zeros16, stage_v)
    for k in range(STRIPE // 128):
        pltpu.sync_copy(stage_v, acc.at[pl.ds(s * STRIPE + k * 128, 128)])
    pltpu.sync_copy(dst2d.at[pl.ds(w * CPT_DEG, CPT_DEG)], idx_v)
    plsc.subcore_barrier()
    pltpu.sync_copy(ones16, stage_v)

    def body(j, carry):
        pltpu.sync_copy(stage_v, acc.at[idx_v.at[j]], add=True)
        return carry

    lax.fori_loop(0, CPT_DEG, body, 0)
    plsc.subcore_barrier()
    pltpu.sync_copy(acc.at[pl.ds(s * STRIPE, STRIPE)],
                    out.at[c].at[pl.ds(s * STRIPE, STRIPE)])


_NBUF = 3
_DEPTH = 2
_CH = 64
_NCHP = 160
_PIECES = 2
_ASTRIPE = NPA // 16


@functools.partial(
    pl.kernel,
    out_type=jax.ShapeDtypeStruct((2, NPA, 128), jnp.float32),
    mesh=_MESH,
    compiler_params=_SC_PARAMS,
    scratch_types=[
        pltpu.VMEM((_NCHP, _CH), jnp.int32),
        pltpu.VMEM((_NCHP, _CH), jnp.int32),
        [pltpu.VMEM((_CH, 128), jnp.float32)] * _NBUF,
        [pltpu.SemaphoreType.DMA] * _NBUF,
        [pltpu.SemaphoreType.DMA] * _NBUF,
        pltpu.VMEM_SHARED((NPA, 128), jnp.float32),
    ],
)
def _agg_sc(y_st, src64, dst64, zeros_c, out, src_v, dst_v, sts, sgs,
            sss, acc):
    c = lax.axis_index("c")
    s = lax.axis_index("s")
    pltpu.sync_copy(zeros_c, sts[0])
    for k in range(9):
        pltpu.sync_copy(sts[0], acc.at[pl.ds(s * _ASTRIPE + k * _CH, _CH)])
    pltpu.sync_copy(sts[0].at[pl.ds(0, 56)],
                    acc.at[pl.ds(s * _ASTRIPE + 576, 56)])
    plsc.subcore_barrier()
    ytab = y_st.at[c]

    def wait1(buf, sem):
        pltpu.make_async_copy(zeros_c, buf, sem).wait()

    for pc in range(_PIECES):
        base = s * (_PIECES * _NCHP) + pc * _NCHP
        pltpu.sync_copy(src64.at[pl.ds(base, _NCHP)], src_v)
        pltpu.sync_copy(dst64.at[pl.ds(base, _NCHP)], dst_v)
        for b in range(_DEPTH):
            pltpu.async_copy(ytab.at[src_v.at[b]], sts[b], sgs[b])

        def body(i, carry):
            for t in range(_NBUF):
                j = i * _NBUF + t
                wait1(sts[t], sgs[t])
                pltpu.async_copy(sts[t], acc.at[dst_v.at[j]],
                                 sss[t], add=True)
                b2 = (t + _DEPTH) % _NBUF

                @pl.when(j >= _NBUF - _DEPTH)
                def _drain():
                    wait1(sts[b2], sss[b2])

                @pl.when(j + _DEPTH < _NCHP)
                def _refill():
                    pltpu.async_copy(ytab.at[src_v.at[j + _DEPTH]],
                                     sts[b2], sgs[b2])

            return carry

        _NFULL = _NCHP // _NBUF
        lax.fori_loop(0, _NFULL, body, 0)
        for j in range(_NFULL * _NBUF, _NCHP):
            t = j % _NBUF
            wait1(sts[t], sgs[t])
            pltpu.async_copy(sts[t], acc.at[dst_v.at[j]], sss[t], add=True)
            wait1(sts[(t + _DEPTH) % _NBUF], sss[(t + _DEPTH) % _NBUF])
        for k in range(_NCHP - (_NBUF - _DEPTH), _NCHP):
            wait1(sts[k % _NBUF], sss[k % _NBUF])
    plsc.subcore_barrier()
    pltpu.sync_copy(acc.at[pl.ds(s * _ASTRIPE, _ASTRIPE)],
                    out.at[c].at[pl.ds(s * _ASTRIPE, _ASTRIPE)])


_BLK = 400
_GRID = N // _BLK


def _dinv_of(d0, d1):
    deg = d0[:, :1] + d1[:, :1] + 1.0
    return lax.rsqrt(deg)


def _tc1_body(x_ref, w_ref, d0_ref, d1_ref, y_ref):
    dinv = _dinv_of(d0_ref[...], d1_ref[...])
    y = jnp.dot(x_ref[...], w_ref[...], precision=_HIGH,
                preferred_element_type=jnp.float32) * dinv
    y_ref[0] = y[:, :128]
    y_ref[1] = y[:, 128:]


def _ln_lrelu(pre, g, be):
    m = jnp.mean(pre, axis=1, keepdims=True)
    cz = pre - m
    var = jnp.mean(cz * cz, axis=1, keepdims=True)
    ln = cz * lax.rsqrt(var + 1e-5) * g + be
    return jnp.where(ln >= 0, ln, 0.2 * ln)


def _tc2_body(a_ref, y_ref, d0_ref, d1_ref, b_ref, g_ref, be_ref, w2_ref,
              out_ref):
    dinv = _dinv_of(d0_ref[...], d1_ref[...])
    agg = jnp.concatenate([a_ref[0], a_ref[1]], axis=1)
    y = jnp.concatenate([y_ref[0], y_ref[1]], axis=1)
    pre = (agg + y) * dinv + b_ref[...]
    h = _ln_lrelu(pre, g_ref[...], be_ref[...])
    xw = jnp.dot(h, w2_ref[...], precision=_HIGH,
                 preferred_element_type=jnp.float32) * dinv
    out_ref[0] = xw[:, :128]
    out_ref[1] = xw[:, 128:]


def _tc3_body(a_ref, y_ref, d0_ref, d1_ref, b_ref, g_ref, be_ref,
              wa1_ref, ba1_ref, wa2_ref, ba2_ref, h_ref, s_ref):
    dinv = _dinv_of(d0_ref[...], d1_ref[...])
    agg = jnp.concatenate([a_ref[0], a_ref[1]], axis=1)
    y = jnp.concatenate([y_ref[0], y_ref[1]], axis=1)
    pre = (agg + y) * dinv + b_ref[...]
    h = _ln_lrelu(pre, g_ref[...], be_ref[...])
    h_ref[...] = h
    t = jnp.tanh(jnp.dot(h, wa1_ref[...], precision=_HIGH,
                         preferred_element_type=jnp.float32) + ba1_ref[...])
    s_ref[...] = jnp.dot(t, wa2_ref[...], precision=_HIGH,
                         preferred_element_type=jnp.float32) + ba2_ref[...]


def _tc4_body(s_ref, h_ref, att_ref, gf_ref):
    sc = s_ref[...]
    m = jnp.max(sc)
    e = jnp.exp(sc - m)
    z = jnp.sum(e)
    att = e / z
    att_ref[...] = att
    gf_ref[...] = lax.dot_general(att, h_ref[...], (((0,), (0,)), ((), ())),
                                  precision=_HIGH,
                                  preferred_element_type=jnp.float32)


def _row_spec(width):
    return pl.BlockSpec((_BLK, width), lambda i: (i, 0))


def _full_spec(shape):
    nd = len(shape)
    return pl.BlockSpec(shape, lambda i: (0,) * nd)


def _split_spec():
    return pl.BlockSpec((2, _BLK, 128), lambda i: (0, i, 0))


def kernel(x, edge_index, W1, b1, g1, be1, W2, b2, g2, be2, Wa1, ba1, Wa2, ba2):
    e = edge_index.shape[1]
    pad = E_PAD - e
    src = edge_index[0].astype(jnp.int32)
    dst = edge_index[1].astype(jnp.int32)
    src_p = jnp.concatenate([src, jnp.zeros((pad,), jnp.int32)])
    dst_p = jnp.concatenate([dst, jnp.full((pad,), N, jnp.int32)])
    src64 = src_p.reshape(E_PAD // 64, 64)
    dst64 = dst_p.reshape(E_PAD // 64, 64)
    dst2d = dst_p.reshape(ROWS, 128)
    zeros16 = jnp.zeros((128, 16), jnp.float32)
    ones16 = jnp.ones((128, 16), jnp.float32)
    zeros_c = jnp.zeros((64, 128), jnp.float32)
    b1r, g1r, be1r = b1.reshape(1, -1), g1.reshape(1, -1), be1.reshape(1, -1)
    b2r, g2r, be2r = b2.reshape(1, -1), g2.reshape(1, -1), be2.reshape(1, -1)
    ba1r, ba2r = ba1.reshape(1, -1), ba2.reshape(1, -1)

    deg2 = _deg_sc(dst2d, zeros16, ones16)
    deg0 = deg2[0, :N]
    deg1 = deg2[1, :N]

    y1 = pl.pallas_call(
        _tc1_body,
        grid=(_GRID,),
        in_specs=[_row_spec(D_IN), _full_spec((D_IN, D_H)),
                  _row_spec(16), _row_spec(16)],
        out_specs=_split_spec(),
        out_shape=jax.ShapeDtypeStruct((2, N, 128), jnp.float32),
    )(x, W1, deg0, deg1)

    agg1 = _agg_sc(y1, src64, dst64, zeros_c)

    y2 = pl.pallas_call(
        _tc2_body,
        grid=(_GRID,),
        in_specs=[_split_spec(), _split_spec(), _row_spec(16), _row_spec(16),
                  _full_spec((1, D_H)), _full_spec((1, D_H)),
                  _full_spec((1, D_H)), _full_spec((D_H, D_H))],
        out_specs=_split_spec(),
        out_shape=jax.ShapeDtypeStruct((2, N, 128), jnp.float32),
    )(agg1, y1, deg0, deg1, b1r, g1r, be1r, W2)

    agg2 = _agg_sc(y2, src64, dst64, zeros_c)

    h2, s = pl.pallas_call(
        _tc3_body,
        grid=(_GRID,),
        in_specs=[_split_spec(), _split_spec(), _row_spec(16), _row_spec(16),
                  _full_spec((1, D_H)), _full_spec((1, D_H)),
                  _full_spec((1, D_H)), _full_spec((D_H, D_H)),
                  _full_spec((1, D_H)), _full_spec((D_H, 1)),
                  _full_spec((1, 1))],
        out_specs=[_row_spec(D_H), _row_spec(1)],
        out_shape=[jax.ShapeDtypeStruct((N, D_H), jnp.float32),
                   jax.ShapeDtypeStruct((N, 1), jnp.float32)],
    )(agg2, y2, deg0, deg1, b2r, g2r, be2r, Wa1, ba1r, Wa2, ba2r)

    att, gf = pl.pallas_call(
        _tc4_body,
        grid=(1,),
        in_specs=[_full_spec((N, 1)), _full_spec((N, D_H))],
        out_specs=[_full_spec((N, 1)), _full_spec((1, D_H))],
        out_shape=[jax.ShapeDtypeStruct((N, 1), jnp.float32),
                   jax.ShapeDtypeStruct((1, D_H), jnp.float32)],
    )(s, h2)

    return gf, att.reshape(N)
